# baseline (speedup 1.0000x reference)
import functools

import jax
import jax.numpy as jnp
from jax import lax
from jax.experimental import pallas as pl
from jax.experimental.pallas import tpu as pltpu
from jax.experimental.pallas import tpu_sc as plsc

NUM_CLASS = 100000
BATCH = 4096
CTX_DIM = 512
N_CLS_CTX = 4
SEQ_LEN = 77
PRE = 5
SUF = SEQ_LEN - PRE - N_CLS_CTX
ROW = N_CLS_CTX * CTX_DIM
OUT_W = SEQ_LEN * CTX_DIM
MID_OFF = PRE * CTX_DIM

_INFO = plsc.get_sparse_core_info()
_NC, _NS = _INFO.num_cores, _INFO.num_subcores
_NW = _NC * _NS
B_PER_W = BATCH // _NW
CHUNK = 32
N_CHUNKS = B_PER_W // CHUNK


def _sc_gather_into(label, cls2d):
    mesh = plsc.VectorSubcoreMesh(core_axis_name="c", subcore_axis_name="s")

    @functools.partial(
        pl.kernel,
        mesh=mesh,
        out_type=jax.ShapeDtypeStruct((BATCH, OUT_W), jnp.float32),
        scratch_types=[
            pltpu.VMEM((CHUNK,), jnp.int32),
            pltpu.VMEM((CHUNK, ROW), jnp.float32),
            pltpu.SemaphoreType.DMA,
        ],
    )
    def k(cls_hbm, label_hbm, out_hbm, idx_v, rows_v, sem):
        wid = lax.axis_index("s") * _NC + lax.axis_index("c")
        base = wid * B_PER_W
        for c in range(N_CHUNKS):
            off = base + c * CHUNK
            pltpu.sync_copy(label_hbm.at[pl.ds(off, CHUNK)], idx_v)
            pltpu.async_copy(cls_hbm.at[idx_v], rows_v, sem).wait()
            pltpu.sync_copy(
                rows_v, out_hbm.at[pl.ds(off, CHUNK), pl.ds(MID_OFF, ROW)])

    return k(cls2d, label)


BBLK = 64


def _tc_broadcast(combo, out1):
    def body(combo_ref, alias_ref, out_ref):
        del alias_ref
        out_ref[...] = jnp.broadcast_to(combo_ref[0], (BBLK, CTX_DIM))

    return pl.pallas_call(
        body,
        grid=(BATCH // BBLK, PRE + SUF),
        in_specs=[
            pl.BlockSpec((1, 1, CTX_DIM), lambda i, j: (j, 0, 0)),
            pl.BlockSpec(memory_space=pl.ANY),
        ],
        out_specs=pl.BlockSpec(
            (BBLK, CTX_DIM),
            lambda i, j: (i, jnp.where(j < PRE, j, j + N_CLS_CTX))),
        out_shape=jax.ShapeDtypeStruct((BATCH, OUT_W), jnp.float32),
        input_output_aliases={1: 0},
    )(combo, out1)


def kernel(label, cls_ctx, token_prefix, token_suffix):
    cls2d = cls_ctx.reshape(NUM_CLASS, ROW)
    out1 = _sc_gather_into(label.astype(jnp.int32), cls2d)
    combo = jnp.concatenate([token_prefix[0], token_suffix[0]], axis=0)
    out2d = _tc_broadcast(combo.reshape(PRE + SUF, 1, CTX_DIM), out1)
    return out2d.reshape(BATCH, SEQ_LEN, CTX_DIM)
